# baseline (speedup 1.0000x reference)
import functools

import jax
import jax.numpy as jnp
from jax import lax
from jax.experimental import pallas as pl
from jax.experimental.pallas import tpu as pltpu
from jax.experimental.pallas import tpu_sc as plsc

_NUM_FIELDS = 26
_FIELD_DIM = 100000
_B = 16384
_NC = 2
_NS = 16
_NW = _NC * _NS
_BPW = _B // _NW
_CHUNK = 128
_NJ = _BPW // _CHUNK
_L = 16


def _tec_body(x_hbm, w_hbm, b_hbm, out_hbm, x_v, val_v, acc_v, z_v, b_v, sem):
    wid = lax.axis_index("s") * _NC + lax.axis_index("c")
    base = wid * _BPW

    pltpu.sync_copy(x_hbm.at[wid], x_v)

    def _zero(c, carry):
        z_v[pl.ds(c * _L, _L)] = jnp.zeros((_L,), jnp.int32)
        return carry

    lax.fori_loop(0, _CHUNK // _L, _zero, 0)

    for f in range(_NUM_FIELDS):
        off = jnp.int32(f * _FIELD_DIM)
        for j in range(_NJ):

            def _add(c, carry, f=f, j=j, off=off):
                sl = pl.ds(c * _L, _L)
                x_v[f, j, sl] = x_v[f, j, sl] + off
                return carry

            lax.fori_loop(0, _CHUNK // _L, _add, 0)

    pltpu.make_async_copy(b_hbm.at[z_v], b_v, sem).start()
    for f in range(_NUM_FIELDS):
        for j in range(_NJ):
            pltpu.make_async_copy(
                w_hbm.at[x_v.at[f, j]], val_v.at[f, j], sem
            ).start()
    pltpu.make_async_copy(b_hbm.at[z_v], b_v, sem).wait()
    for f in range(_NUM_FIELDS):
        for j in range(_NJ):
            pltpu.make_async_copy(
                w_hbm.at[x_v.at[f, j]], val_v.at[f, j], sem
            ).wait()

    for j in range(_NJ):

        def _red(c, carry, j=j):
            sl = pl.ds(c * _L, _L)
            acc = b_v[sl]
            for f in range(_NUM_FIELDS):
                acc = acc + val_v[f, j, sl]
            acc_v[pl.ds(j * _CHUNK + c * _L, _L)] = acc
            return carry

        lax.fori_loop(0, _CHUNK // _L, _red, 0)

    pltpu.sync_copy(acc_v, out_hbm.at[pl.ds(base, _BPW)])


_lookup = functools.partial(
    pl.kernel,
    out_type=jax.ShapeDtypeStruct((_B,), jnp.float32),
    mesh=plsc.VectorSubcoreMesh(
        core_axis_name="c", subcore_axis_name="s", num_cores=_NC
    ),
    scratch_types=[
        pltpu.VMEM((_NUM_FIELDS, _NJ, _CHUNK), jnp.int32),
        pltpu.VMEM((_NUM_FIELDS, _NJ, _CHUNK), jnp.float32),
        pltpu.VMEM((_BPW,), jnp.float32),
        pltpu.VMEM((_CHUNK,), jnp.int32),
        pltpu.VMEM((_CHUNK,), jnp.float32),
        pltpu.SemaphoreType.DMA,
    ],
)(_tec_body)


@jax.jit
def kernel(x, W, bias):
    xt = (
        x.T.reshape(_NUM_FIELDS, _NW, _BPW)
        .transpose(1, 0, 2)
        .reshape(_NW, _NUM_FIELDS, _NJ, _CHUNK)
    )
    out = _lookup(xt, W.reshape(-1), bias)
    return out[:, None]


# speedup vs baseline: 1.0145x; 1.0145x over previous
import functools

import jax
import jax.numpy as jnp
from jax import lax
from jax.experimental import pallas as pl
from jax.experimental.pallas import tpu as pltpu
from jax.experimental.pallas import tpu_sc as plsc

_NUM_FIELDS = 26
_FIELD_DIM = 100000
_B = 16384
_NC = 2
_NS = 16
_NW = _NC * _NS
_BPW = _B // _NW
_CHUNK = 128
_NJ = _BPW // _CHUNK
_L = 16
_SLAB = _BPW * _NUM_FIELDS


def _tec_body(x_hbm, w_hbm, out_hbm, slab_v, idx_v, val_v, acc_v, sem):
    wid = lax.axis_index("s") * _NC + lax.axis_index("c")
    base = wid * _BPW

    pltpu.sync_copy(x_hbm.at[wid], slab_v)

    i26 = lax.iota(jnp.int32, _L) * jnp.int32(_NUM_FIELDS)
    for f in range(_NUM_FIELDS):
        off = jnp.int32(f * _FIELD_DIM)
        for j in range(_NJ):

            def _mk(c, carry, f=f, j=j, off=off):
                srcbase = (j * _CHUNK) * _NUM_FIELDS + f + c * (_L * _NUM_FIELDS)
                g = plsc.load_gather(slab_v, [i26 + srcbase])
                idx_v[f, j, pl.ds(c * _L, _L)] = g + off
                return carry

            lax.fori_loop(0, _CHUNK // _L, _mk, 0)

    for f in range(_NUM_FIELDS):
        for j in range(_NJ):
            pltpu.make_async_copy(
                w_hbm.at[idx_v.at[f, j]], val_v.at[f, j], sem
            ).start()
    for f in range(_NUM_FIELDS):
        for j in range(_NJ):
            pltpu.make_async_copy(
                w_hbm.at[idx_v.at[f, j]], val_v.at[f, j], sem
            ).wait()

    for j in range(_NJ):

        def _red(c, carry, j=j):
            sl = pl.ds(c * _L, _L)
            acc = val_v[0, j, sl]
            for f in range(1, _NUM_FIELDS):
                acc = acc + val_v[f, j, sl]
            acc_v[pl.ds(j * _CHUNK + c * _L, _L)] = acc
            return carry

        lax.fori_loop(0, _CHUNK // _L, _red, 0)

    pltpu.sync_copy(acc_v, out_hbm.at[pl.ds(base, _BPW)])


_lookup = functools.partial(
    pl.kernel,
    out_type=jax.ShapeDtypeStruct((_B,), jnp.float32),
    mesh=plsc.VectorSubcoreMesh(
        core_axis_name="c", subcore_axis_name="s", num_cores=_NC
    ),
    compiler_params=pltpu.CompilerParams(needs_layout_passes=False),
    scratch_types=[
        pltpu.VMEM((_SLAB,), jnp.int32),
        pltpu.VMEM((_NUM_FIELDS, _NJ, _CHUNK), jnp.int32),
        pltpu.VMEM((_NUM_FIELDS, _NJ, _CHUNK), jnp.float32),
        pltpu.VMEM((_BPW,), jnp.float32),
        pltpu.SemaphoreType.DMA,
    ],
)(_tec_body)


@jax.jit
def kernel(x, W, bias):
    out = _lookup(x.reshape(_NW, _SLAB), W.reshape(-1))
    return out[:, None] + bias[None, :]


# speedup vs baseline: 1.1322x; 1.1160x over previous
import functools

import jax
import jax.numpy as jnp
import numpy as np
from jax import lax
from jax.experimental import pallas as pl
from jax.experimental.pallas import tpu as pltpu
from jax.experimental.pallas import tpu_sc as plsc

_NUM_FIELDS = 26
_FIELD_DIM = 100000
_B = 16384
_NC = 2
_NS = 16
_NW = _NC * _NS
_BPW = _B // _NW
_CHUNK = 128
_NJ = _BPW // _CHUNK
_L = 16

_OFFSETS = np.arange(_NUM_FIELDS, dtype=np.int32) * _FIELD_DIM


def _tec_body(x_hbm, w_hbm, out_hbm, idx_v, val_v, acc_v, sem):
    wid = lax.axis_index("s") * _NC + lax.axis_index("c")
    base = wid * _BPW

    pltpu.sync_copy(x_hbm.at[wid], idx_v)

    cp = pltpu.make_async_copy(w_hbm.at[idx_v], val_v, sem)
    cp.start()
    cp.wait()

    def _red(c, carry):
        acc = val_v[pl.ds(c * _L, _L)]
        for f in range(1, _NUM_FIELDS):
            acc = acc + val_v[pl.ds(f * _BPW + c * _L, _L)]
        acc_v[pl.ds(c * _L, _L)] = acc
        return carry

    lax.fori_loop(0, _BPW // _L, _red, 0)

    pltpu.sync_copy(acc_v, out_hbm.at[pl.ds(base, _BPW)])


_lookup = functools.partial(
    pl.kernel,
    out_type=jax.ShapeDtypeStruct((_B,), jnp.float32),
    mesh=plsc.VectorSubcoreMesh(
        core_axis_name="c", subcore_axis_name="s", num_cores=_NC
    ),
    scratch_types=[
        pltpu.VMEM((_NUM_FIELDS * _BPW,), jnp.int32),
        pltpu.VMEM((_NUM_FIELDS * _BPW,), jnp.float32),
        pltpu.VMEM((_BPW,), jnp.float32),
        pltpu.SemaphoreType.DMA,
    ],
)(_tec_body)


@jax.jit
def kernel(x, W, bias):
    xt = (
        (x + jnp.asarray(_OFFSETS)[None, :])
        .T.reshape(_NUM_FIELDS, _NW, _BPW)
        .transpose(1, 0, 2)
        .reshape(_NW, _NUM_FIELDS * _BPW)
    )
    out = _lookup(xt, W.reshape(-1))
    return out[:, None] + bias[None, :]
